# baseline (speedup 1.0000x reference)
import jax
import jax.numpy as jnp
from jax import lax
from jax.experimental import pallas as pl
from jax.experimental.pallas import tpu as pltpu
from jax.experimental.pallas import tpu_sc as plsc

NUM_CORES = 2
NUM_SUBCORES = 16
NW = NUM_CORES * NUM_SUBCORES

B = 16384
D = 64
V = 1000000
B_PER_W = B // NW
GRP = 16
N_GRP = B_PER_W // GRP
K = 8
PANEL_W = 128
LAST_PANEL = V // PANEL_W
TAIL_COL = LAST_PANEL * PANEL_W
TAIL_W = V - TAIL_COL
OUT_LAG = 4


def _vextract(v16, lane):
    g = jnp.take_along_axis(v16, jnp.full((GRP,), lane, jnp.int32), axis=0)
    return g[0]


def _emb_kernel(xs_hbm, pos_hbm, wt_hbm, out_hbm,
                xs_v, pos_v, ring_v, tail_v, row_v, sem_p, sem_o):
    wid = lax.axis_index("s") * NUM_CORES + lax.axis_index("c")
    n0 = wid * B_PER_W
    pltpu.sync_copy(xs_hbm.at[pl.ds(n0, B_PER_W)], xs_v)
    pltpu.sync_copy(pos_hbm.at[pl.ds(n0, B_PER_W)], pos_v)
    pltpu.sync_copy(wt_hbm.at[:, pl.ds(TAIL_COL, TAIL_W)], tail_v)

    def read_xs(m):
        mbase = pl.multiple_of((m // GRP) * GRP, GRP)
        return _vextract(xs_v[pl.ds(mbase, GRP)], m % GRP)

    def fire(mp, lp, fire_idx):
        mp2 = lax.while_loop(
            lambda m: jnp.logical_and(m < B_PER_W - 1,
                                      read_xs(m) // PANEL_W <= lp),
            lambda m: m + 1, mp)
        p = jnp.minimum(read_xs(mp2) // PANEL_W, LAST_PANEL - 1)
        slot = fire_idx % K
        col = pl.multiple_of(p * PANEL_W, PANEL_W)
        pltpu.async_copy(
            wt_hbm.at[pl.ds(0, D // 2), pl.ds(col, PANEL_W)],
            ring_v.at[pl.ds(pl.multiple_of(slot * D, 8), D // 2), :], sem_p)
        pltpu.async_copy(
            wt_hbm.at[pl.ds(D // 2, D // 2), pl.ds(col, PANEL_W)],
            ring_v.at[pl.ds(pl.multiple_of(slot * D + D // 2, 8), D // 2), :],
            sem_p)
        return mp2, p

    mp, lp = jnp.int32(0), jnp.int32(-1)
    for u in range(K - 1):
        mp, lp = fire(mp, lp, jnp.int32(u))

    def wait_panel():
        for _ in range(2):
            pltpu.make_async_copy(
                wt_hbm.at[pl.ds(0, D // 2), pl.ds(0, PANEL_W)],
                ring_v.at[pl.ds(0, D // 2), :], sem_p).wait()

    def wait_row():
        pltpu.make_async_copy(
            out_hbm.at[pl.ds(0, D)], row_v.at[pl.ds(0, D)], sem_o).wait()

    def group_body(g, carry):
        t, p_cur, mp, lp = carry

        @pl.when(g >= OUT_LAG)
        def _():
            for _ in range(GRP):
                wait_row()

        xs16 = xs_v[pl.ds(pl.multiple_of(g * GRP, GRP), GRP)]
        pos16 = pos_v[pl.ds(pl.multiple_of(g * GRP, GRP), GRP)]
        half = (g % OUT_LAG) * (GRP * D)

        def advance(t, mp, lp):
            t2 = t + 1
            mp2, lp2 = fire(mp, lp, t2 + K - 1)
            wait_panel()
            return t2, mp2, lp2

        def no_advance(t, mp, lp):
            return t, mp, lp

        def body(carry, with_tail):
            t, p_cur, mp, lp = carry
            for k in range(GRP):
                i = xs16[k]
                p = i // PANEL_W
                r = i % PANEL_W
                lvec = jnp.full((GRP,), r, jnp.int32)
                if with_tail:
                    is_tail = p >= LAST_PANEL
                    do_adv = jnp.logical_and(p_cur != p,
                                             jnp.logical_not(is_tail))
                else:
                    do_adv = p_cur != p
                t, mp, lp = lax.cond(do_adv, advance, no_advance, t, mp, lp)
                p_cur = jnp.where(is_tail, p_cur, p) if with_tail else p
                slot = t % K
                for m in range(D // GRP):
                    dvec = lax.iota(jnp.int32, GRP) + m * GRP
                    vals = plsc.load_gather(ring_v, [slot * D + dvec, lvec])
                    if with_tail:
                        vals_tail = plsc.load_gather(tail_v, [dvec, lvec])
                        vals = jnp.where(is_tail, vals_tail, vals)
                    row_v[pl.ds(pl.multiple_of(half + k * D + m * GRP, GRP),
                                GRP)] = vals
                pos = pos16[k]
                pltpu.async_copy(
                    row_v.at[pl.ds(pl.multiple_of(half + k * D, GRP), D)],
                    out_hbm.at[pl.ds(pl.multiple_of(pos * D, GRP), D)],
                    sem_o)
            return t, p_cur, mp, lp

        has_tail = xs16[GRP - 1] >= TAIL_COL
        return lax.cond(has_tail,
                        lambda c: body(c, True),
                        lambda c: body(c, False),
                        (t, p_cur, mp, lp))

    lax.fori_loop(0, N_GRP, group_body,
                  (jnp.int32(-1), jnp.int32(-1), mp, lp))

    for _ in range(K - 1):
        wait_panel()
    for _ in range(OUT_LAG * GRP):
        wait_row()


@jax.jit
def kernel(x, weight):
    xi = x.astype(jnp.int32)
    pos = jnp.argsort(xi).astype(jnp.int32)
    xs = jnp.sort(xi)
    wt = weight.T
    mesh = plsc.VectorSubcoreMesh(
        core_axis_name="c", subcore_axis_name="s",
        num_cores=NUM_CORES, num_subcores=NUM_SUBCORES,
    )
    out_flat = pl.kernel(
        _emb_kernel,
        out_type=jax.ShapeDtypeStruct((B * D,), jnp.float32),
        mesh=mesh,
        scratch_types=[
            pltpu.VMEM((B_PER_W,), jnp.int32),
            pltpu.VMEM((B_PER_W,), jnp.int32),
            pltpu.VMEM((K * D, PANEL_W), jnp.float32),
            pltpu.VMEM((D, TAIL_W), jnp.float32),
            pltpu.VMEM((OUT_LAG * GRP * D,), jnp.float32),
            pltpu.SemaphoreType.DMA,
            pltpu.SemaphoreType.DMA,
        ],
        compiler_params=pltpu.CompilerParams(needs_layout_passes=False),
    )(xs, pos, wt)
    return out_flat.reshape(B, D)


# speedup vs baseline: 1.0055x; 1.0055x over previous
import jax
import jax.numpy as jnp
from jax import lax
from jax.experimental import pallas as pl
from jax.experimental.pallas import tpu as pltpu
from jax.experimental.pallas import tpu_sc as plsc

NUM_CORES = 2
NUM_SUBCORES = 16
NW = NUM_CORES * NUM_SUBCORES

B = 16384
D = 64
V = 1000000
B_PER_W = B // NW
GRP = 16
N_GRP = B_PER_W // GRP
K = 8
PANEL_W = 128
LAST_PANEL = V // PANEL_W
TAIL_COL = LAST_PANEL * PANEL_W
TAIL_W = V - TAIL_COL
OUT_LAG = 4


def _vextract(v16, lane):
    g = jnp.take_along_axis(v16, jnp.full((GRP,), lane, jnp.int32), axis=0)
    return g[0]


def _emb_kernel(xs_hbm, pos_hbm, wt_hbm, out_hbm,
                xs_v, pos_v, ring_v, tail_v, row_v, sem_p, sem_o):
    wid = lax.axis_index("s") * NUM_CORES + lax.axis_index("c")
    n0 = wid * B_PER_W
    pltpu.sync_copy(xs_hbm.at[pl.ds(n0, B_PER_W)], xs_v)
    pltpu.sync_copy(pos_hbm.at[pl.ds(n0, B_PER_W)], pos_v)
    pltpu.sync_copy(wt_hbm.at[:, pl.ds(TAIL_COL, TAIL_W)], tail_v)

    def read_xs(m):
        mbase = pl.multiple_of((m // GRP) * GRP, GRP)
        return _vextract(xs_v[pl.ds(mbase, GRP)], m % GRP)

    def fire(mp, lp, fire_idx):
        mp2 = lax.while_loop(
            lambda m: jnp.logical_and(m < B_PER_W - 1,
                                      read_xs(m) // PANEL_W <= lp),
            lambda m: m + 1, mp)
        p = jnp.minimum(read_xs(mp2) // PANEL_W, LAST_PANEL - 1)
        slot = fire_idx % K
        pltpu.async_copy(
            wt_hbm.at[:, pl.ds(pl.multiple_of(p * PANEL_W, PANEL_W), PANEL_W)],
            ring_v.at[pl.ds(pl.multiple_of(slot * D, 8), D), :], sem_p)
        return mp2, p

    mp, lp = jnp.int32(0), jnp.int32(-1)
    for u in range(K - 1):
        mp, lp = fire(mp, lp, jnp.int32(u))

    def wait_panel():
        pltpu.make_async_copy(
            wt_hbm.at[:, pl.ds(0, PANEL_W)],
            ring_v.at[pl.ds(0, D), :], sem_p).wait()

    def wait_row():
        pltpu.make_async_copy(
            out_hbm.at[pl.ds(0, D)], row_v.at[pl.ds(0, D)], sem_o).wait()

    def group_body(g, carry):
        t, p_cur, mp, lp = carry

        @pl.when(g >= OUT_LAG)
        def _():
            for _ in range(GRP):
                wait_row()

        xs16 = xs_v[pl.ds(pl.multiple_of(g * GRP, GRP), GRP)]
        pos16 = pos_v[pl.ds(pl.multiple_of(g * GRP, GRP), GRP)]
        half = (g % OUT_LAG) * (GRP * D)

        def advance(t, mp, lp):
            t2 = t + 1
            wait_panel()
            mp2, lp2 = fire(mp, lp, t2 + K - 1)
            return t2, mp2, lp2

        def no_advance(t, mp, lp):
            return t, mp, lp

        def body(carry, with_tail):
            t, p_cur, mp, lp = carry
            for k in range(GRP):
                i = xs16[k]
                p = i // PANEL_W
                r = i % PANEL_W
                lvec = jnp.full((GRP,), r, jnp.int32)
                if with_tail:
                    is_tail = p >= LAST_PANEL
                    do_adv = jnp.logical_and(p_cur != p,
                                             jnp.logical_not(is_tail))
                else:
                    do_adv = p_cur != p
                t, mp, lp = lax.cond(do_adv, advance, no_advance, t, mp, lp)
                p_cur = jnp.where(is_tail, p_cur, p) if with_tail else p
                slot = t % K
                for m in range(D // GRP):
                    dvec = lax.iota(jnp.int32, GRP) + m * GRP
                    vals = plsc.load_gather(ring_v, [slot * D + dvec, lvec])
                    if with_tail:
                        vals_tail = plsc.load_gather(tail_v, [dvec, lvec])
                        vals = jnp.where(is_tail, vals_tail, vals)
                    row_v[pl.ds(pl.multiple_of(half + k * D + m * GRP, GRP),
                                GRP)] = vals
                pos = pos16[k]
                pltpu.async_copy(
                    row_v.at[pl.ds(pl.multiple_of(half + k * D, GRP), D)],
                    out_hbm.at[pl.ds(pl.multiple_of(pos * D, GRP), D)],
                    sem_o)
            return t, p_cur, mp, lp

        has_tail = xs16[GRP - 1] >= TAIL_COL
        return lax.cond(has_tail,
                        lambda c: body(c, True),
                        lambda c: body(c, False),
                        (t, p_cur, mp, lp))

    lax.fori_loop(0, N_GRP, group_body,
                  (jnp.int32(-1), jnp.int32(-1), mp, lp))

    for _ in range(K - 1):
        wait_panel()
    for _ in range(OUT_LAG * GRP):
        wait_row()


@jax.jit
def kernel(x, weight):
    xi = x.astype(jnp.int32)
    pos = jnp.argsort(xi).astype(jnp.int32)
    xs = jnp.sort(xi)
    wt = weight.T
    mesh = plsc.VectorSubcoreMesh(
        core_axis_name="c", subcore_axis_name="s",
        num_cores=NUM_CORES, num_subcores=NUM_SUBCORES,
    )
    out_flat = pl.kernel(
        _emb_kernel,
        out_type=jax.ShapeDtypeStruct((B * D,), jnp.float32),
        mesh=mesh,
        scratch_types=[
            pltpu.VMEM((B_PER_W,), jnp.int32),
            pltpu.VMEM((B_PER_W,), jnp.int32),
            pltpu.VMEM((K * D, PANEL_W), jnp.float32),
            pltpu.VMEM((D, TAIL_W), jnp.float32),
            pltpu.VMEM((OUT_LAG * GRP * D,), jnp.float32),
            pltpu.SemaphoreType.DMA,
            pltpu.SemaphoreType.DMA,
        ],
        compiler_params=pltpu.CompilerParams(needs_layout_passes=False),
    )(xs, pos, wt)
    return out_flat.reshape(B, D)
